# baseline (speedup 1.0000x reference)
import functools

import jax
import jax.numpy as jnp
from jax import lax
from jax.experimental import pallas as pl
from jax.experimental.pallas import tpu as pltpu
from jax.experimental.pallas import tpu_sc as plsc

NN = 10000
NE = 320000
DIN = 128
DHID = 128
DOUT = 32

NC = 2
NS = 16
K = 128
NBUF = 4
SDEPTH = 2

NCH = 160
_E_TILE = NCH * K
_PAD = NS * _E_TILE - NE

NCHD = 80
_PAD_D = NS * NCHD * K - NE // NC

_DR_T = 10
_ROWS_T = NN // _DR_T

_MESH = plsc.VectorSubcoreMesh(
    core_axis_name="c", subcore_axis_name="s", num_cores=NC, num_subcores=NS
)
_SC_PARAMS = pltpu.CompilerParams(use_tc_tiling_on_sc=False)


@functools.partial(
    pl.kernel,
    out_type=jax.ShapeDtypeStruct((NC, NN, 16), jnp.float32),
    mesh=_MESH,
    scratch_types=[
        pltpu.VMEM((NCHD, K), jnp.int32),
        pltpu.VMEM((K, 16), jnp.float32),
        pltpu.VMEM_SHARED((NN + 8, 16), jnp.float32),
        pltpu.SemaphoreType.DMA,
    ],
    compiler_params=_SC_PARAMS,
)
def _deg_kernel(dstp_hbm, zeros_hbm, ones_hbm, degp_hbm, dstv, onesv, shared, sem):
    cid = lax.axis_index("c")
    sid = lax.axis_index("s")

    @pl.when(sid < _DR_T)
    def _init():
        pltpu.sync_copy(zeros_hbm, shared.at[pl.ds(sid * _ROWS_T, _ROWS_T)])

    pltpu.sync_copy(ones_hbm, onesv)
    pltpu.sync_copy(dstp_hbm.at[pl.ds((cid * NS + sid) * NCHD, NCHD)], dstv)
    plsc.subcore_barrier()

    def body(i, carry):
        pltpu.async_copy(onesv, shared.at[dstv.at[i]], sem, add=True)

        @pl.when(i > 0)
        def _():
            pltpu.make_async_copy(onesv, shared.at[dstv.at[i - 1]], sem).wait()

        return carry

    lax.fori_loop(0, NCHD, body, 0)
    pltpu.make_async_copy(onesv, shared.at[dstv.at[NCHD - 1]], sem).wait()
    plsc.subcore_barrier()

    @pl.when(sid < _DR_T)
    def _drain():
        stripe = pl.ds(sid * _ROWS_T, _ROWS_T)
        pltpu.sync_copy(shared.at[stripe], degp_hbm.at[cid, stripe])


def _make_agg_kernel(d2):
    @functools.partial(
        pl.kernel,
        out_type=jax.ShapeDtypeStruct((NC, NN, d2), jnp.float32),
        mesh=_MESH,
        scratch_types=[
            pltpu.VMEM((NCH, K), jnp.int32),
            pltpu.VMEM((NCH, K), jnp.int32),
            [pltpu.VMEM((K, d2), jnp.float32) for _ in range(NBUF)],
            pltpu.VMEM_SHARED((NN + 8, d2), jnp.float32),
            pltpu.SemaphoreType.DMA,
            pltpu.SemaphoreType.DMA,
        ],
        compiler_params=_SC_PARAMS,
    )
    def _agg(y_hbm, srcp_hbm, dstp_hbm, zeros_hbm, aggp_hbm,
             srcv, dstv, bufs, shared, sem_g, sem_s):
        cid = lax.axis_index("c")
        sid = lax.axis_index("s")

        @pl.when(sid < _DR_T)
        def _init():
            pltpu.sync_copy(zeros_hbm, shared.at[pl.ds(sid * _ROWS_T, _ROWS_T)])

        pltpu.sync_copy(srcp_hbm.at[pl.ds((cid * NS + sid) * NCH, NCH)], srcv)
        pltpu.sync_copy(dstp_hbm.at[pl.ds(sid * NCH, NCH)], dstv)
        plsc.subcore_barrier()

        for b in range(NBUF):
            pltpu.async_copy(y_hbm.at[srcv.at[b]], bufs[b], sem_g)

        def outer(g, carry):
            for b in range(NBUF):
                i = g * NBUF + b
                pltpu.make_async_copy(y_hbm.at[srcv.at[i]], bufs[b], sem_g).wait()
                pltpu.async_copy(bufs[b], shared.at[dstv.at[i]], sem_s, add=True)

                bprev = (b - SDEPTH) % NBUF

                def _retire_and_refill():
                    pltpu.make_async_copy(
                        bufs[bprev], shared.at[dstv.at[i - SDEPTH]], sem_s
                    ).wait()

                    @pl.when(i - SDEPTH + NBUF < NCH)
                    def _():
                        pltpu.async_copy(
                            y_hbm.at[srcv.at[i - SDEPTH + NBUF]], bufs[bprev],
                            sem_g,
                        )

                if b < SDEPTH:
                    pl.when(i >= SDEPTH)(_retire_and_refill)
                else:
                    _retire_and_refill()
            return carry

        lax.fori_loop(0, NCH // NBUF, outer, 0)
        for j in range(NCH - SDEPTH, NCH):
            pltpu.make_async_copy(
                bufs[j % NBUF], shared.at[dstv.at[j]], sem_s
            ).wait()
        plsc.subcore_barrier()

        @pl.when(sid < _DR_T)
        def _drain():
            stripe = pl.ds(sid * _ROWS_T, _ROWS_T)
            pltpu.sync_copy(shared.at[stripe], aggp_hbm.at[cid, stripe])

    return _agg


_agg64 = _make_agg_kernel(64)
_agg16 = _make_agg_kernel(16)


_RB = 1000


def _pre_body(x_ref, w_ref, da_ref, db_ref, ys_ref, dis_ref):
    deg = da_ref[0][:, 0:1] + db_ref[0][:, 0:1] + 1.0
    dis = lax.rsqrt(deg)
    xw = jnp.dot(x_ref[:], w_ref[:], preferred_element_type=jnp.float32)
    y = xw * dis
    ys_ref[0] = y[:, :64]
    ys_ref[1] = y[:, 64:]
    dis_ref[:] = jnp.broadcast_to(dis, (_RB, 8))


def _pre_kernel(x, w1, degp):
    return pl.pallas_call(
        _pre_body,
        grid=(NN // _RB,),
        in_specs=[
            pl.BlockSpec((_RB, DIN), lambda i: (i, 0)),
            pl.BlockSpec((DIN, DHID), lambda i: (0, 0)),
            pl.BlockSpec((1, _RB, 16), lambda i: (0, i, 0)),
            pl.BlockSpec((1, _RB, 16), lambda i: (1, i, 0)),
        ],
        out_specs=[
            pl.BlockSpec((2, _RB, 64), lambda i: (0, i, 0)),
            pl.BlockSpec((_RB, 8), lambda i: (i, 0)),
        ],
        out_shape=[
            jax.ShapeDtypeStruct((2, NN, 64), jnp.float32),
            jax.ShapeDtypeStruct((NN, 8), jnp.float32),
        ],
    )(x, w1, degp, degp)


def _mid_body(aa_ref, ab_ref, ys_ref, dis_ref, b_ref, w_ref, y2s_ref):
    dis = dis_ref[:, 0:1]
    agg = jnp.concatenate([aa_ref[0], ab_ref[0]], axis=1)
    y = jnp.concatenate([ys_ref[0], ys_ref[1]], axis=1)
    h = jnp.maximum(dis * (agg + y) + b_ref[:], 0.0)
    y2 = dis * jnp.dot(h, w_ref[:], preferred_element_type=jnp.float32)
    y2s_ref[0] = y2[:, :16]
    y2s_ref[1] = y2[:, 16:]


def _mid_kernel(aggp1, y1s, dis8, b1, w2):
    return pl.pallas_call(
        _mid_body,
        grid=(NN // _RB,),
        in_specs=[
            pl.BlockSpec((1, _RB, 64), lambda i: (0, i, 0)),
            pl.BlockSpec((1, _RB, 64), lambda i: (1, i, 0)),
            pl.BlockSpec((2, _RB, 64), lambda i: (0, i, 0)),
            pl.BlockSpec((_RB, 8), lambda i: (i, 0)),
            pl.BlockSpec((1, DHID), lambda i: (0, 0)),
            pl.BlockSpec((DHID, DOUT), lambda i: (0, 0)),
        ],
        out_specs=pl.BlockSpec((2, _RB, 16), lambda i: (0, i, 0)),
        out_shape=jax.ShapeDtypeStruct((2, NN, 16), jnp.float32),
    )(aggp1, aggp1, y1s, dis8, b1, w2)


def _final_body(aa_ref, ab_ref, y2s_ref, dis_ref, b_ref, out_ref):
    dis = dis_ref[:, 0:1]
    agg = jnp.concatenate([aa_ref[0], ab_ref[0]], axis=1)
    y2 = jnp.concatenate([y2s_ref[0], y2s_ref[1]], axis=1)
    z = dis * (agg + y2) + b_ref[:]
    m = jnp.max(z, axis=1, keepdims=True)
    s = jnp.sum(jnp.exp(z - m), axis=1, keepdims=True)
    out_ref[:] = z - m - jnp.log(s)


def _final_kernel(aggp2, y2s, dis8, b2):
    return pl.pallas_call(
        _final_body,
        grid=(NN // _RB,),
        in_specs=[
            pl.BlockSpec((1, _RB, 16), lambda i: (0, i, 0)),
            pl.BlockSpec((1, _RB, 16), lambda i: (1, i, 0)),
            pl.BlockSpec((2, _RB, 16), lambda i: (0, i, 0)),
            pl.BlockSpec((_RB, 8), lambda i: (i, 0)),
            pl.BlockSpec((1, DOUT), lambda i: (0, 0)),
        ],
        out_specs=pl.BlockSpec((_RB, DOUT), lambda i: (i, 0)),
        out_shape=jax.ShapeDtypeStruct((NN, DOUT), jnp.float32),
    )(aggp2, aggp2, y2s, dis8, b2)


def kernel(x, W1, b1, W2, b2, edge_index):
    src = edge_index[0].astype(jnp.int32)
    dst = edge_index[1].astype(jnp.int32)

    zpad = jnp.zeros((_PAD,), jnp.int32)
    tpad = jnp.full((_PAD,), NN, jnp.int32)
    srcp = jnp.concatenate(
        [src, zpad, src + NN, zpad]).reshape(NC * NS * NCH, K)
    dstp = jnp.concatenate([dst, tpad]).reshape(NS * NCH, K)

    zpd = jnp.full((_PAD_D,), NN, jnp.int32)
    dstpd = jnp.concatenate(
        [dst[: NE // NC], zpd, dst[NE // NC:], zpd]).reshape(NC * NS * NCHD, K)

    zeros16 = jnp.zeros((_ROWS_T, 16), jnp.float32)
    zeros64 = jnp.zeros((_ROWS_T, 64), jnp.float32)
    ones16 = jnp.ones((K, 16), jnp.float32)

    degp = _deg_kernel(dstpd, zeros16, ones16)
    y1s, dis8 = _pre_kernel(x, W1, degp)

    aggp1 = _agg64(y1s.reshape(2 * NN, 64), srcp, dstp, zeros64)
    y2s = _mid_kernel(aggp1, y1s, dis8, b1[None, :], W2)

    aggp2 = _agg16(y2s.reshape(2 * NN, 16), srcp, dstp, zeros16)
    return _final_kernel(aggp2, y2s, dis8, b2[None, :])


# speedup vs baseline: 1.1030x; 1.1030x over previous
import functools

import jax
import jax.numpy as jnp
from jax import lax
from jax.experimental import pallas as pl
from jax.experimental.pallas import tpu as pltpu
from jax.experimental.pallas import tpu_sc as plsc

NN = 10000
NE = 320000
DIN = 128
DHID = 128
DOUT = 32

NC = 2
NS = 16
K = 128
NBUF = 4
SDEPTH = 1

NCH = 160
_E_TILE = NCH * K
_PAD = NS * _E_TILE - NE

NCHD = 80
_PAD_D = NS * NCHD * K - NE // NC

_DR_T = 10
_ROWS_T = NN // _DR_T

_MESH = plsc.VectorSubcoreMesh(
    core_axis_name="c", subcore_axis_name="s", num_cores=NC, num_subcores=NS
)
_SC_PARAMS = pltpu.CompilerParams(use_tc_tiling_on_sc=False)


@functools.partial(
    pl.kernel,
    out_type=jax.ShapeDtypeStruct((NC, NN, 16), jnp.float32),
    mesh=_MESH,
    scratch_types=[
        pltpu.VMEM((NCHD, K), jnp.int32),
        pltpu.VMEM((K, 16), jnp.float32),
        pltpu.VMEM_SHARED((NN + 8, 16), jnp.float32),
        pltpu.SemaphoreType.DMA,
    ],
    compiler_params=_SC_PARAMS,
)
def _deg_kernel(dstp_hbm, zeros_hbm, ones_hbm, degp_hbm, dstv, onesv, shared, sem):
    cid = lax.axis_index("c")
    sid = lax.axis_index("s")

    @pl.when(sid < _DR_T)
    def _init():
        pltpu.sync_copy(zeros_hbm, shared.at[pl.ds(sid * _ROWS_T, _ROWS_T)])

    pltpu.sync_copy(ones_hbm, onesv)
    pltpu.sync_copy(dstp_hbm.at[pl.ds((cid * NS + sid) * NCHD, NCHD)], dstv)
    plsc.subcore_barrier()

    def body(i, carry):
        pltpu.async_copy(onesv, shared.at[dstv.at[i]], sem, add=True)

        @pl.when(i > 0)
        def _():
            pltpu.make_async_copy(onesv, shared.at[dstv.at[i - 1]], sem).wait()

        return carry

    lax.fori_loop(0, NCHD, body, 0)
    pltpu.make_async_copy(onesv, shared.at[dstv.at[NCHD - 1]], sem).wait()
    plsc.subcore_barrier()

    @pl.when(sid < _DR_T)
    def _drain():
        stripe = pl.ds(sid * _ROWS_T, _ROWS_T)
        pltpu.sync_copy(shared.at[stripe], degp_hbm.at[cid, stripe])


def _make_agg_kernel(d2, nch):
    @functools.partial(
        pl.kernel,
        out_type=jax.ShapeDtypeStruct((NC, NN, d2), jnp.float32),
        mesh=_MESH,
        scratch_types=[
            pltpu.VMEM((nch, K), jnp.int32),
            pltpu.VMEM((nch, K), jnp.int32),
            [pltpu.VMEM((K, d2), jnp.float32) for _ in range(NBUF)],
            pltpu.VMEM_SHARED((NN + 8, d2), jnp.float32),
            pltpu.SemaphoreType.DMA,
            pltpu.SemaphoreType.DMA,
        ],
        compiler_params=_SC_PARAMS,
    )
    def _agg(y_hbm, srcp_hbm, dstp_hbm, zeros_hbm, aggp_hbm,
             srcv, dstv, bufs, shared, sem_g, sem_s):
        cid = lax.axis_index("c")
        sid = lax.axis_index("s")

        @pl.when(sid < _DR_T)
        def _init():
            pltpu.sync_copy(zeros_hbm, shared.at[pl.ds(sid * _ROWS_T, _ROWS_T)])

        row0 = (cid * NS + sid) * nch
        pltpu.sync_copy(srcp_hbm.at[pl.ds(row0, nch)], srcv)
        pltpu.sync_copy(dstp_hbm.at[pl.ds(row0, nch)], dstv)
        plsc.subcore_barrier()

        for b in range(NBUF):
            pltpu.async_copy(y_hbm.at[srcv.at[b]], bufs[b], sem_g)

        def outer(g, carry):
            for b in range(NBUF):
                i = g * NBUF + b
                pltpu.make_async_copy(y_hbm.at[srcv.at[i]], bufs[b], sem_g).wait()
                pltpu.async_copy(bufs[b], shared.at[dstv.at[i]], sem_s, add=True)

                bprev = (b - SDEPTH) % NBUF

                def _retire_and_refill():
                    pltpu.make_async_copy(
                        bufs[bprev], shared.at[dstv.at[i - SDEPTH]], sem_s
                    ).wait()

                    @pl.when(i - SDEPTH + NBUF < nch)
                    def _():
                        pltpu.async_copy(
                            y_hbm.at[srcv.at[i - SDEPTH + NBUF]], bufs[bprev],
                            sem_g,
                        )

                if b < SDEPTH:
                    pl.when(i >= SDEPTH)(_retire_and_refill)
                else:
                    _retire_and_refill()
            return carry

        lax.fori_loop(0, nch // NBUF, outer, 0)
        for j in range(nch - SDEPTH, nch):
            pltpu.make_async_copy(
                bufs[j % NBUF], shared.at[dstv.at[j]], sem_s
            ).wait()
        plsc.subcore_barrier()

        @pl.when(sid < _DR_T)
        def _drain():
            stripe = pl.ds(sid * _ROWS_T, _ROWS_T)
            pltpu.sync_copy(shared.at[stripe], aggp_hbm.at[cid, stripe])

    return _agg


_agg64 = _make_agg_kernel(64, NCH)
_agg32 = _make_agg_kernel(32, NCHD)


_RB = 1000


def _pre_body(x_ref, w_ref, da_ref, db_ref, ys_ref, dis_ref):
    deg = da_ref[0][:, 0:1] + db_ref[0][:, 0:1] + 1.0
    dis = lax.rsqrt(deg)
    xw = jnp.dot(x_ref[:], w_ref[:], preferred_element_type=jnp.float32)
    y = xw * dis
    ys_ref[0] = y[:, :64]
    ys_ref[1] = y[:, 64:]
    dis_ref[:] = jnp.broadcast_to(dis, (_RB, 8))


def _pre_kernel(x, w1, degp):
    return pl.pallas_call(
        _pre_body,
        grid=(NN // _RB,),
        in_specs=[
            pl.BlockSpec((_RB, DIN), lambda i: (i, 0)),
            pl.BlockSpec((DIN, DHID), lambda i: (0, 0)),
            pl.BlockSpec((1, _RB, 16), lambda i: (0, i, 0)),
            pl.BlockSpec((1, _RB, 16), lambda i: (1, i, 0)),
        ],
        out_specs=[
            pl.BlockSpec((2, _RB, 64), lambda i: (0, i, 0)),
            pl.BlockSpec((_RB, 8), lambda i: (i, 0)),
        ],
        out_shape=[
            jax.ShapeDtypeStruct((2, NN, 64), jnp.float32),
            jax.ShapeDtypeStruct((NN, 8), jnp.float32),
        ],
    )(x, w1, degp, degp)


def _mid_body(aa_ref, ab_ref, ys_ref, dis_ref, b_ref, w_ref, y2_ref):
    dis = dis_ref[:, 0:1]
    agg = jnp.concatenate([aa_ref[0], ab_ref[0]], axis=1)
    y = jnp.concatenate([ys_ref[0], ys_ref[1]], axis=1)
    h = jnp.maximum(dis * (agg + y) + b_ref[:], 0.0)
    y2_ref[:] = dis * jnp.dot(h, w_ref[:], preferred_element_type=jnp.float32)


def _mid_kernel(aggp1, y1s, dis8, b1, w2):
    return pl.pallas_call(
        _mid_body,
        grid=(NN // _RB,),
        in_specs=[
            pl.BlockSpec((1, _RB, 64), lambda i: (0, i, 0)),
            pl.BlockSpec((1, _RB, 64), lambda i: (1, i, 0)),
            pl.BlockSpec((2, _RB, 64), lambda i: (0, i, 0)),
            pl.BlockSpec((_RB, 8), lambda i: (i, 0)),
            pl.BlockSpec((1, DHID), lambda i: (0, 0)),
            pl.BlockSpec((DHID, DOUT), lambda i: (0, 0)),
        ],
        out_specs=pl.BlockSpec((_RB, DOUT), lambda i: (i, 0)),
        out_shape=jax.ShapeDtypeStruct((NN, DOUT), jnp.float32),
    )(aggp1, aggp1, y1s, dis8, b1, w2)


def _final_body(aa_ref, ab_ref, y2_ref, dis_ref, b_ref, out_ref):
    dis = dis_ref[:, 0:1]
    agg = aa_ref[0] + ab_ref[0]
    z = dis * (agg + y2_ref[:]) + b_ref[:]
    m = jnp.max(z, axis=1, keepdims=True)
    s = jnp.sum(jnp.exp(z - m), axis=1, keepdims=True)
    out_ref[:] = z - m - jnp.log(s)


def _final_kernel(aggp2, y2, dis8, b2):
    return pl.pallas_call(
        _final_body,
        grid=(NN // _RB,),
        in_specs=[
            pl.BlockSpec((1, _RB, DOUT), lambda i: (0, i, 0)),
            pl.BlockSpec((1, _RB, DOUT), lambda i: (1, i, 0)),
            pl.BlockSpec((_RB, DOUT), lambda i: (i, 0)),
            pl.BlockSpec((_RB, 8), lambda i: (i, 0)),
            pl.BlockSpec((1, DOUT), lambda i: (0, 0)),
        ],
        out_specs=pl.BlockSpec((_RB, DOUT), lambda i: (i, 0)),
        out_shape=jax.ShapeDtypeStruct((NN, DOUT), jnp.float32),
    )(aggp2, aggp2, y2, dis8, b2)


def kernel(x, W1, b1, W2, b2, edge_index):
    src = edge_index[0].astype(jnp.int32)
    dst = edge_index[1].astype(jnp.int32)

    zpad = jnp.zeros((_PAD,), jnp.int32)
    tpad = jnp.full((_PAD,), NN, jnp.int32)
    srcp = jnp.concatenate(
        [src, zpad, src + NN, zpad]).reshape(NC * NS * NCH, K)
    dstp = jnp.concatenate(
        [dst, tpad, dst, tpad]).reshape(NC * NS * NCH, K)

    zpd = jnp.full((_PAD_D,), NN, jnp.int32)
    spd = jnp.zeros((_PAD_D,), jnp.int32)
    dstpd = jnp.concatenate(
        [dst[: NE // NC], zpd, dst[NE // NC:], zpd]).reshape(NC * NS * NCHD, K)
    srcpd = jnp.concatenate(
        [src[: NE // NC], spd, src[NE // NC:], spd]).reshape(NC * NS * NCHD, K)

    zeros16 = jnp.zeros((_ROWS_T, 16), jnp.float32)
    zeros32 = jnp.zeros((_ROWS_T, 32), jnp.float32)
    zeros64 = jnp.zeros((_ROWS_T, 64), jnp.float32)
    ones16 = jnp.ones((K, 16), jnp.float32)

    degp = _deg_kernel(dstpd, zeros16, ones16)
    y1s, dis8 = _pre_kernel(x, W1, degp)

    aggp1 = _agg64(y1s.reshape(2 * NN, 64), srcp, dstp, zeros64)
    y2 = _mid_kernel(aggp1, y1s, dis8, b1[None, :], W2)

    aggp2 = _agg32(y2, srcpd, dstpd, zeros32)
    return _final_kernel(aggp2, y2, dis8, b2[None, :])
